# baseline (speedup 1.0000x reference)
import jax
import jax.numpy as jnp
from jax.experimental import pallas as pl
from jax.experimental.pallas import tpu as pltpu

S, B, D = 2048, 1, 1024
LATENT, E, TOPK, EH, SH = 256, 16, 2, 512, 1024
AUX = 1e-4
N = S * B
BT = 1024
IB = N // BT


def _nt_dot(a, b, precision=jax.lax.Precision.DEFAULT):
    return jax.lax.dot_general(
        a, b, (((1,), (1,)), ((), ())),
        precision=precision, preferred_element_type=jnp.float32)


def _moe_kernel(h_ref, norm_w_ref, ld_ref, lu_ref, rw_ref, rb_ref,
                fc1_ref, vth_ref, fc2_ref, sfc1_ref, svth_ref, sfc2_ref,
                sgw_ref, opw_ref, out_ref, cnt_ref, psum_ref,
                w_full_s, latb_s, shared_s, acc_s):
    hi = jax.lax.Precision.HIGHEST
    bf = jnp.bfloat16
    i = pl.program_id(0)
    j = pl.program_id(1)

    @pl.when(j == 0)
    def _prologue():
        x3 = h_ref[...]
        ms = jnp.mean(x3 * x3, axis=(1, 2), keepdims=True)
        hn = (x3 * jax.lax.rsqrt(ms + 1e-6)).reshape(BT, D) * norm_w_ref[...]
        logits = _nt_dot(hn, rw_ref[...], hi) + rb_ref[...]
        scores = jax.nn.sigmoid(logits)
        gsc = jax.nn.sigmoid(_nt_dot(hn, sgw_ref[...], hi))
        hnb = hn.astype(bf)
        col = jax.lax.broadcasted_iota(jnp.int32, (BT, E), 1)
        m1 = jnp.max(scores, axis=-1, keepdims=True)
        i1 = jnp.argmax(scores, axis=-1)[:, None]
        masked = jnp.where(col == i1, -jnp.inf, scores)
        m2 = jnp.max(masked, axis=-1, keepdims=True)
        i2 = jnp.argmax(masked, axis=-1)[:, None]
        denom = m1 + m2 + 1e-8
        w_full_s[...] = jnp.where(col == i1, m1 / denom, 0.0) + jnp.where(
            col == i2, m2 / denom, 0.0)
        sel = (col == i1).astype(jnp.float32) + (col == i2).astype(jnp.float32)

        @pl.when(i == 0)
        def _():
            cnt_ref[...] = jnp.zeros_like(cnt_ref)
            psum_ref[...] = jnp.zeros_like(psum_ref)

        cnt_ref[...] += jnp.sum(sel, axis=0, keepdims=True)
        psum_ref[...] += jnp.sum(scores, axis=0, keepdims=True)

        latb_s[...] = _nt_dot(hnb, ld_ref[...].astype(bf)).astype(bf)
        acc_s[...] = jnp.zeros_like(acc_s)

        sgate = _nt_dot(hnb, sfc1_ref[:SH].astype(bf))
        svth = svth_ref[...]
        sg = jnp.where(sgate >= svth, svth, 0.0).astype(bf)
        slin = _nt_dot(hnb, sfc1_ref[SH:].astype(bf)).astype(bf)
        sact = sg * slin
        shared = _nt_dot(sact, sfc2_ref[...].astype(bf))
        shared_s[...] = (shared * gsc).astype(bf)

    @pl.when(j >= 1)
    def _expert():
        e = j - 1
        latb = latb_s[...]
        g = _nt_dot(latb, fc1_ref[0].astype(bf))
        gate = g[:, :EH]
        lin = g[:, EH:]
        vth = vth_ref[pl.ds(e, 1)]
        act = jnp.where(gate >= vth, vth, 0.0) * lin
        eo = _nt_dot(act.astype(bf), fc2_ref[0].astype(bf))
        col = jax.lax.broadcasted_iota(jnp.int32, (BT, E), 1)
        wcol = jnp.sum(
            jnp.where(col == e, w_full_s[...], 0.0), axis=-1, keepdims=True)
        acc_s[...] += eo * wcol

    @pl.when(j == E)
    def _finalize():
        routed = _nt_dot(acc_s[...].astype(bf), lu_ref[...].astype(bf))
        pre = routed + shared_s[...].astype(jnp.float32)
        final = _nt_dot(pre.astype(bf), opw_ref[...].astype(bf))
        out_ref[...] = h_ref[...] + final.reshape(BT, 8, 128)


@jax.jit
def kernel(h, norm_w, latent_down_W, latent_up_W, router_W, router_bias,
           expert_fc1_W, expert_vth, expert_fc2_W, shared_fc1_W, shared_vth,
           shared_fc2_W, shared_gate_W, out_proj_W):
    hf = h.reshape(N, 8, 128)
    full = lambda *shape: pl.BlockSpec(shape, lambda i, j: (0,) * len(shape))
    ej = lambda i, j: (jnp.maximum(j - 1, 0), 0, 0)
    out, cnt, psum = pl.pallas_call(
        _moe_kernel,
        grid=(IB, E + 1),
        in_specs=[
            pl.BlockSpec((BT, 8, 128), lambda i, j: (i, 0, 0)),
            full(1, D),
            full(LATENT, D),
            full(D, LATENT),
            full(E, D),
            full(1, E),
            pl.BlockSpec((1, 2 * EH, LATENT), ej),
            full(E, EH),
            pl.BlockSpec((1, LATENT, EH), ej),
            full(2 * SH, D),
            full(1, SH),
            full(D, SH),
            full(1, D),
            full(D, D),
        ],
        out_specs=[
            pl.BlockSpec((BT, 8, 128), lambda i, j: (i, 0, 0)),
            pl.BlockSpec((1, E), lambda i, j: (0, 0)),
            pl.BlockSpec((1, E), lambda i, j: (0, 0)),
        ],
        out_shape=[
            jax.ShapeDtypeStruct((N, 8, 128), jnp.float32),
            jax.ShapeDtypeStruct((1, E), jnp.float32),
            jax.ShapeDtypeStruct((1, E), jnp.float32),
        ],
        scratch_shapes=[
            pltpu.VMEM((BT, E), jnp.float32),
            pltpu.VMEM((BT, LATENT), jnp.bfloat16),
            pltpu.VMEM((BT, D), jnp.bfloat16),
            pltpu.VMEM((BT, LATENT), jnp.float32),
        ],
    )(hf, norm_w.reshape(1, D), latent_down_W, latent_up_W, router_W,
      router_bias.reshape(1, E), expert_fc1_W, expert_vth, expert_fc2_W,
      shared_fc1_W, shared_vth.reshape(1, SH), shared_fc2_W, shared_gate_W,
      out_proj_W)
    lb_loss = E * jnp.sum((cnt[0] / N) * (psum[0] / N)) * AUX
    return out.reshape(S, B, D), lb_loss


# speedup vs baseline: 1.2384x; 1.2384x over previous
import jax
import jax.numpy as jnp
from jax.experimental import pallas as pl

S, B, D = 2048, 1, 1024
LATENT, E, TOPK, EH, SH = 256, 16, 2, 512, 1024
AUX = 1e-4
N = S * B
BT = 256
GRID = N // BT


def _nt_dot(a, b, precision=jax.lax.Precision.DEFAULT):
    return jax.lax.dot_general(
        a, b, (((1,), (1,)), ((), ())),
        precision=precision, preferred_element_type=jnp.float32)


def _moe_kernel(h_ref, norm_w_ref, ld_ref, lu_ref, rw_ref, rb_ref,
                fc1_ref, vth_ref, fc2_ref, sfc1_ref, svth_ref, sfc2_ref,
                sgw_ref, opw_ref, out_ref, cnt_ref, psum_ref):
    bf = jnp.bfloat16
    step = pl.program_id(0)
    x3 = h_ref[...]
    ms = jnp.mean(x3 * x3, axis=(1, 2), keepdims=True)
    hn = (x3 * jax.lax.rsqrt(ms + 1e-6)).reshape(BT, D) * norm_w_ref[...]
    logits = _nt_dot(hn, rw_ref[...]) + rb_ref[...]
    scores = jax.nn.sigmoid(logits)
    col = jax.lax.broadcasted_iota(jnp.int32, (BT, E), 1)
    m1 = jnp.max(scores, axis=-1, keepdims=True)
    i1 = jnp.argmax(scores, axis=-1)[:, None]
    masked = jnp.where(col == i1, -jnp.inf, scores)
    m2 = jnp.max(masked, axis=-1, keepdims=True)
    i2 = jnp.argmax(masked, axis=-1)[:, None]
    denom = m1 + m2 + 1e-8
    w_full = jnp.where(col == i1, m1 / denom, 0.0) + jnp.where(
        col == i2, m2 / denom, 0.0)
    sel = (col == i1).astype(jnp.float32) + (col == i2).astype(jnp.float32)

    hnb = hn.astype(bf)

    latent = _nt_dot(hnb, ld_ref[...].astype(bf))
    latb = latent.astype(bf)

    acc = jnp.zeros((BT, LATENT), jnp.float32)
    for e in range(E):
        g = _nt_dot(latb, fc1_ref[e].astype(bf))
        gate = g[:, :EH]
        lin = g[:, EH:]
        vth = vth_ref[e][None, :]
        act = jnp.where(gate >= vth, vth, 0.0) * lin
        eo = _nt_dot(act.astype(bf), fc2_ref[e].astype(bf))
        acc = acc + eo * w_full[:, e][:, None]
    routed = _nt_dot(acc.astype(bf), lu_ref[...].astype(bf))

    s = _nt_dot(hnb, sfc1_ref[...].astype(bf))
    sgate = s[:, :SH]
    slin = s[:, SH:]
    svth = svth_ref[...]
    sact = jnp.where(sgate >= svth, svth, 0.0) * slin
    shared = _nt_dot(sact.astype(bf), sfc2_ref[...].astype(bf))
    gsc = jax.nn.sigmoid(
        jnp.sum(hn * sgw_ref[...], axis=-1, keepdims=True))
    shared = shared * gsc

    final = _nt_dot((routed + shared).astype(bf), opw_ref[...].astype(bf))
    out_ref[...] = x3 + final.reshape(BT, 8, 128)

    @pl.when(step == 0)
    def _():
        cnt_ref[...] = jnp.zeros_like(cnt_ref)
        psum_ref[...] = jnp.zeros_like(psum_ref)

    cnt_ref[...] += jnp.sum(sel, axis=0, keepdims=True)
    psum_ref[...] += jnp.sum(scores, axis=0, keepdims=True)


@jax.jit
def kernel(h, norm_w, latent_down_W, latent_up_W, router_W, router_bias,
           expert_fc1_W, expert_vth, expert_fc2_W, shared_fc1_W, shared_vth,
           shared_fc2_W, shared_gate_W, out_proj_W):
    hf = h.reshape(N, 8, 128)
    full = lambda *shape: pl.BlockSpec(shape, lambda i: (0,) * len(shape))
    out, cnt, psum = pl.pallas_call(
        _moe_kernel,
        grid=(GRID,),
        in_specs=[
            pl.BlockSpec((BT, 8, 128), lambda i: (i, 0, 0)),
            full(1, D),
            full(LATENT, D),
            full(D, LATENT),
            full(E, D),
            full(1, E),
            full(E, 2 * EH, LATENT),
            full(E, EH),
            full(E, LATENT, EH),
            full(2 * SH, D),
            full(1, SH),
            full(D, SH),
            full(1, D),
            full(D, D),
        ],
        out_specs=[
            pl.BlockSpec((BT, 8, 128), lambda i: (i, 0, 0)),
            pl.BlockSpec((1, E), lambda i: (0, 0)),
            pl.BlockSpec((1, E), lambda i: (0, 0)),
        ],
        out_shape=[
            jax.ShapeDtypeStruct((N, 8, 128), jnp.float32),
            jax.ShapeDtypeStruct((1, E), jnp.float32),
            jax.ShapeDtypeStruct((1, E), jnp.float32),
        ],
    )(hf, norm_w.reshape(1, D), latent_down_W, latent_up_W, router_W,
      router_bias.reshape(1, E), expert_fc1_W, expert_vth, expert_fc2_W,
      shared_fc1_W, shared_vth.reshape(1, SH), shared_fc2_W, shared_gate_W,
      out_proj_W)
    lb_loss = E * jnp.sum((cnt[0] / N) * (psum[0] / N)) * AUX
    return out.reshape(S, B, D), lb_loss


# speedup vs baseline: 1.4070x; 1.1361x over previous
import jax
import jax.numpy as jnp
from jax.experimental import pallas as pl

S, B, D = 2048, 1, 1024
LATENT, E, TOPK, EH, SH = 256, 16, 2, 512, 1024
AUX = 1e-4
N = S * B
BT = 512
GRID = N // BT


def _nt_dot(a, b, precision=jax.lax.Precision.DEFAULT):
    return jax.lax.dot_general(
        a, b, (((1,), (1,)), ((), ())),
        precision=precision, preferred_element_type=jnp.float32)


def _moe_kernel(h_ref, norm_w_ref, ld_ref, lu_ref, rw_ref, rb_ref,
                fc1_ref, vth_ref, fc2_ref, sfc1_ref, svth_ref, sfc2_ref,
                sgw_ref, opw_ref, out_ref, cnt_ref, psum_ref):
    bf = jnp.bfloat16
    step = pl.program_id(0)
    x3 = h_ref[...]
    ms = jnp.mean(x3 * x3, axis=(1, 2), keepdims=True)
    hn = (x3 * jax.lax.rsqrt(ms + 1e-6)).reshape(BT, D) * norm_w_ref[...]
    logits = _nt_dot(hn, rw_ref[...]) + rb_ref[...]
    scores = jax.nn.sigmoid(logits)
    col = jax.lax.broadcasted_iota(jnp.int32, (BT, E), 1)
    m1 = jnp.max(scores, axis=-1, keepdims=True)
    i1 = jnp.argmax(scores, axis=-1)[:, None]
    masked = jnp.where(col == i1, -jnp.inf, scores)
    m2 = jnp.max(masked, axis=-1, keepdims=True)
    i2 = jnp.argmax(masked, axis=-1)[:, None]
    denom = m1 + m2 + 1e-8
    w_full = jnp.where(col == i1, m1 / denom, 0.0) + jnp.where(
        col == i2, m2 / denom, 0.0)
    sel = (col == i1).astype(jnp.float32) + (col == i2).astype(jnp.float32)

    hnb = hn.astype(bf)

    latent = _nt_dot(hnb, ld_ref[...].astype(bf))
    latb = latent.astype(bf)

    acc = jnp.zeros((BT, LATENT), jnp.float32)
    for e in range(E):
        g = _nt_dot(latb, fc1_ref[e].astype(bf))
        gate = g[:, :EH]
        lin = g[:, EH:]
        vth = vth_ref[e][None, :]
        act = jnp.where(gate >= vth, vth, 0.0) * lin
        eo = _nt_dot(act.astype(bf), fc2_ref[e].astype(bf))
        acc = acc + eo * w_full[:, e][:, None]
    routed = _nt_dot(acc.astype(bf), lu_ref[...].astype(bf))

    s = _nt_dot(hnb, sfc1_ref[...].astype(bf))
    sgate = s[:, :SH]
    slin = s[:, SH:]
    svth = svth_ref[...]
    sact = jnp.where(sgate >= svth, svth, 0.0) * slin
    shared = _nt_dot(sact.astype(bf), sfc2_ref[...].astype(bf))
    gsc = jax.nn.sigmoid(
        jnp.sum(hn * sgw_ref[...], axis=-1, keepdims=True))
    shared = shared * gsc

    final = _nt_dot((routed + shared).astype(bf), opw_ref[...].astype(bf))
    out_ref[...] = x3 + final.reshape(BT, 8, 128)

    @pl.when(step == 0)
    def _():
        cnt_ref[...] = jnp.zeros_like(cnt_ref)
        psum_ref[...] = jnp.zeros_like(psum_ref)

    cnt_ref[...] += jnp.sum(sel, axis=0, keepdims=True)
    psum_ref[...] += jnp.sum(scores, axis=0, keepdims=True)


@jax.jit
def kernel(h, norm_w, latent_down_W, latent_up_W, router_W, router_bias,
           expert_fc1_W, expert_vth, expert_fc2_W, shared_fc1_W, shared_vth,
           shared_fc2_W, shared_gate_W, out_proj_W):
    hf = h.reshape(N, 8, 128)
    full = lambda *shape: pl.BlockSpec(shape, lambda i: (0,) * len(shape))
    out, cnt, psum = pl.pallas_call(
        _moe_kernel,
        grid=(GRID,),
        in_specs=[
            pl.BlockSpec((BT, 8, 128), lambda i: (i, 0, 0)),
            full(1, D),
            full(LATENT, D),
            full(D, LATENT),
            full(E, D),
            full(1, E),
            full(E, 2 * EH, LATENT),
            full(E, EH),
            full(E, LATENT, EH),
            full(2 * SH, D),
            full(1, SH),
            full(D, SH),
            full(1, D),
            full(D, D),
        ],
        out_specs=[
            pl.BlockSpec((BT, 8, 128), lambda i: (i, 0, 0)),
            pl.BlockSpec((1, E), lambda i: (0, 0)),
            pl.BlockSpec((1, E), lambda i: (0, 0)),
        ],
        out_shape=[
            jax.ShapeDtypeStruct((N, 8, 128), jnp.float32),
            jax.ShapeDtypeStruct((1, E), jnp.float32),
            jax.ShapeDtypeStruct((1, E), jnp.float32),
        ],
    )(hf, norm_w.reshape(1, D), latent_down_W, latent_up_W, router_W,
      router_bias.reshape(1, E), expert_fc1_W, expert_vth, expert_fc2_W,
      shared_fc1_W, shared_vth.reshape(1, SH), shared_fc2_W, shared_gate_W,
      out_proj_W)
    lb_loss = E * jnp.sum((cnt[0] / N) * (psum[0] / N)) * AUX
    return out.reshape(S, B, D), lb_loss
